# Initial kernel scaffold; baseline (speedup 1.0000x reference)
import jax
import jax.numpy as jnp
from jax.experimental import pallas as pl


def kernel(node_features, edge_index, post_mask, W_enc, b_enc, W_conv, b_conv, W_out, b_out):
    raise NotImplementedError("write your pallas kernel here")


# speedup vs baseline: 8.7039x; 8.7039x over previous
import functools

import jax
import jax.numpy as jnp
from jax import lax
from jax.experimental import pallas as pl
from jax.experimental.pallas import tpu as pltpu
from jax.experimental.pallas import tpu_sc as plsc

NC, NS, LANES = 2, 16, 16
NW = NC * NS
CHUNK = 80

_N, _D, _H, _E = 10000, 128, 32, 320000
_NPAD = 10240
_EPT = _E // NW
_CPT = _EPT // CHUNK
_RPT = _NPAD // NS
_ZR = 128
_PPAD = 5120
_QPT = _PPAD // NW


def _enc_body(x_ref, w_ref, b_ref, o_ref):
    acc = jnp.dot(x_ref[...], w_ref[...], preferred_element_type=jnp.float32)
    o_ref[...] = jnp.maximum(acc + b_ref[...], 0.0)


def _encoder(x, w_t, b2):
    n, d = x.shape
    h = w_t.shape[1]
    bn = 1000
    return pl.pallas_call(
        _enc_body,
        grid=(n // bn,),
        in_specs=[
            pl.BlockSpec((bn, d), lambda i: (i, 0)),
            pl.BlockSpec((d, h), lambda i: (0, 0)),
            pl.BlockSpec((1, h), lambda i: (0, 0)),
        ],
        out_specs=pl.BlockSpec((bn, h), lambda i: (i, 0)),
        out_shape=jax.ShapeDtypeStruct((n, h), jnp.float32),
    )(x, w_t, b2)


_MESH = plsc.VectorSubcoreMesh(core_axis_name="c", subcore_axis_name="s")


@functools.partial(
    pl.kernel,
    out_type=jax.ShapeDtypeStruct((NC, _NPAD, _H), jnp.float32),
    mesh=_MESH,
    scratch_types=[
        pltpu.VMEM((_CPT, CHUNK), jnp.int32),
        pltpu.VMEM((_CPT, CHUNK), jnp.int32),
        pltpu.VMEM((CHUNK, _H), jnp.float32),
        pltpu.VMEM((_ZR, _H), jnp.float32),
        pltpu.VMEM_SHARED((_NPAD, _H), jnp.float32),
        pltpu.SemaphoreType.DMA,
    ],
    compiler_params=pltpu.CompilerParams(use_tc_tiling_on_sc=False),
)
def _sc_messages(emb_hbm, src_hbm, dst_hbm, out_hbm,
                 src_v, dst_v, rows_v, z_v, acc_sh, sem):
    cid = lax.axis_index("c")
    sid = lax.axis_index("s")
    wid = cid * NS + sid

    zero16 = jnp.zeros((16,), jnp.float32)

    def zfill(i, carry):
        z_v[i, pl.ds(0, 16)] = zero16
        z_v[i, pl.ds(16, 16)] = zero16
        return carry

    lax.fori_loop(0, _ZR, zfill, 0)

    def zcopy(r, carry):
        pltpu.sync_copy(z_v, acc_sh.at[pl.ds(sid * _RPT + r * _ZR, _ZR)])
        return carry

    lax.fori_loop(0, _RPT // _ZR, zcopy, 0)
    plsc.subcore_barrier()

    pltpu.sync_copy(src_hbm.at[wid], src_v)
    pltpu.sync_copy(dst_hbm.at[wid], dst_v)

    def edge_chunk(j, carry):
        pltpu.async_copy(emb_hbm.at[src_v.at[j]], rows_v, sem).wait()
        pltpu.sync_copy(rows_v, acc_sh.at[dst_v.at[j]], add=True)
        return carry

    lax.fori_loop(0, _CPT, edge_chunk, 0)
    plsc.subcore_barrier()

    pltpu.sync_copy(acc_sh.at[pl.ds(sid * _RPT, _RPT)],
                    out_hbm.at[cid, pl.ds(sid * _RPT, _RPT)])


def _head_body(e_ref, p0_ref, p1_ref, wc_ref, bc_ref, wo_ref, bo_ref, o_ref):
    h = e_ref[...] + p0_ref[...] + p1_ref[...]
    e2 = jnp.maximum(
        jnp.dot(h, wc_ref[...], preferred_element_type=jnp.float32)
        + bc_ref[...], 0.0)
    z = jnp.dot(e2, wo_ref[...], preferred_element_type=jnp.float32)
    o_ref[...] = jax.nn.sigmoid(z + bo_ref[...])


def _head(node_emb, p0, p1, wc_t, bc2, wo_t, bo2):
    n, h = node_emb.shape
    bn = 1000
    return pl.pallas_call(
        _head_body,
        grid=(n // bn,),
        in_specs=[
            pl.BlockSpec((bn, h), lambda i: (i, 0)),
            pl.BlockSpec((bn, h), lambda i: (i, 0)),
            pl.BlockSpec((bn, h), lambda i: (i, 0)),
            pl.BlockSpec((h, h), lambda i: (0, 0)),
            pl.BlockSpec((1, h), lambda i: (0, 0)),
            pl.BlockSpec((h, 1), lambda i: (0, 0)),
            pl.BlockSpec((1, 1), lambda i: (0, 0)),
        ],
        out_specs=pl.BlockSpec((bn, 1), lambda i: (i, 0)),
        out_shape=jax.ShapeDtypeStruct((n, 1), jnp.float32),
    )(node_emb, p0, p1, wc_t, bc2, wo_t, bo2)


@functools.partial(
    pl.kernel,
    out_type=jax.ShapeDtypeStruct((_PPAD,), jnp.float32),
    mesh=_MESH,
    scratch_types=[
        pltpu.VMEM((_N,), jnp.float32),
        pltpu.VMEM((_QPT,), jnp.int32),
        pltpu.VMEM((_QPT,), jnp.float32),
    ],
    compiler_params=pltpu.CompilerParams(needs_layout_passes=False),
)
def _sc_post_gather(s_hbm, pm_hbm, out_hbm, s_v, pm_v, o_v):
    cid = lax.axis_index("c")
    sid = lax.axis_index("s")
    wid = cid * NS + sid
    pltpu.sync_copy(s_hbm, s_v)
    pltpu.sync_copy(pm_hbm.at[pl.ds(wid * _QPT, _QPT)], pm_v)

    def gloop(i, carry):
        idx = pm_v[pl.ds(i * LANES, LANES)]
        o_v[pl.ds(i * LANES, LANES)] = plsc.load_gather(s_v, [idx])
        return carry

    lax.fori_loop(0, _QPT // LANES, gloop, 0)
    pltpu.sync_copy(o_v, out_hbm.at[pl.ds(wid * _QPT, _QPT)])


def kernel(node_features, edge_index, post_mask,
           W_enc, b_enc, W_conv, b_conv, W_out, b_out):
    n, d = node_features.shape
    h = W_enc.shape[0]
    e = edge_index.shape[1]
    p = post_mask.shape[0]

    node_emb = _encoder(node_features, W_enc.T, b_enc.reshape(1, h))

    src3d = edge_index[0].reshape(NW, _CPT, CHUNK)
    dst3d = edge_index[1].reshape(NW, _CPT, CHUNK)
    partials = _sc_messages(node_emb, src3d, dst3d)

    s = _head(node_emb, partials[0, :n], partials[1, :n],
              W_conv.T, b_conv.reshape(1, h),
              W_out.T, b_out.reshape(1, 1))

    pm_pad = jnp.zeros((_PPAD,), jnp.int32).at[:p].set(post_mask)
    out = _sc_post_gather(s.reshape(-1), pm_pad)
    return out[:p]


# speedup vs baseline: 15.4671x; 1.7770x over previous
import functools

import jax
import jax.numpy as jnp
from jax import lax
from jax.experimental import pallas as pl
from jax.experimental.pallas import tpu as pltpu
from jax.experimental.pallas import tpu_sc as plsc

NC, NS, LANES = 2, 16, 16
NW = NC * NS
CHUNK = 80

_N, _D, _H, _E = 10000, 128, 32, 320000
_NPAD = 10240
_EPT = _E // NW
_CPT = _EPT // CHUNK
_RPT = _NPAD // NS
_ZR = 128
_PPAD = 5120
_QPT = _PPAD // NW
NBUF = 5


def _enc_body(x_ref, w_ref, b_ref, o_ref):
    acc = jnp.dot(x_ref[...], w_ref[...], preferred_element_type=jnp.float32)
    o_ref[...] = jnp.maximum(acc + b_ref[...], 0.0)


def _encoder(x, w_t, b2):
    n, d = x.shape
    h = w_t.shape[1]
    bn = 1000
    return pl.pallas_call(
        _enc_body,
        grid=(n // bn,),
        in_specs=[
            pl.BlockSpec((bn, d), lambda i: (i, 0)),
            pl.BlockSpec((d, h), lambda i: (0, 0)),
            pl.BlockSpec((1, h), lambda i: (0, 0)),
        ],
        out_specs=pl.BlockSpec((bn, h), lambda i: (i, 0)),
        out_shape=jax.ShapeDtypeStruct((n, h), jnp.float32),
    )(x, w_t, b2)


_MESH = plsc.VectorSubcoreMesh(core_axis_name="c", subcore_axis_name="s")


@functools.partial(
    pl.kernel,
    out_type=jax.ShapeDtypeStruct((NC, _NPAD, _H), jnp.float32),
    mesh=_MESH,
    scratch_types=[
        pltpu.VMEM((_CPT, CHUNK), jnp.int32),
        pltpu.VMEM((_CPT, CHUNK), jnp.int32),
        pltpu.VMEM((NBUF, CHUNK, _H), jnp.float32),
        pltpu.VMEM((_ZR, _H), jnp.float32),
        pltpu.VMEM_SHARED((_NPAD, _H), jnp.float32),
        pltpu.SemaphoreType.DMA((NBUF,)),
        pltpu.SemaphoreType.DMA((NBUF,)),
    ],
    compiler_params=pltpu.CompilerParams(use_tc_tiling_on_sc=False),
)
def _sc_messages(emb_hbm, src_hbm, dst_hbm, out_hbm,
                 src_v, dst_v, rows_v, z_v, acc_sh, gsem, ssem):
    cid = lax.axis_index("c")
    sid = lax.axis_index("s")
    wid = cid * NS + sid

    zero16 = jnp.zeros((16,), jnp.float32)

    def zfill(i, carry):
        z_v[i, pl.ds(0, 16)] = zero16
        z_v[i, pl.ds(16, 16)] = zero16
        return carry

    lax.fori_loop(0, _ZR, zfill, 0)

    def zcopy(r, carry):
        pltpu.sync_copy(z_v, acc_sh.at[pl.ds(sid * _RPT + r * _ZR, _ZR)])
        return carry

    lax.fori_loop(0, _RPT // _ZR, zcopy, 0)
    plsc.subcore_barrier()

    pltpu.sync_copy(src_hbm.at[wid], src_v)
    pltpu.sync_copy(dst_hbm.at[wid], dst_v)

    for b in range(NBUF - 1):
        pltpu.async_copy(emb_hbm.at[src_v.at[b]], rows_v.at[b], gsem.at[b])

    def group(g, carry):
        for b in range(NBUF):
            j = g * NBUF + b
            bp = (b - 1) % NBUF
            pltpu.make_async_copy(
                emb_hbm.at[src_v.at[j]], rows_v.at[b], gsem.at[b]).wait()
            pltpu.async_copy(
                rows_v.at[b], acc_sh.at[dst_v.at[j]], ssem.at[b], add=True)

            @pl.when(j >= 1)
            def _wait_prev():
                pltpu.make_async_copy(
                    rows_v.at[bp], acc_sh.at[dst_v.at[j - 1]],
                    ssem.at[bp]).wait()

            @pl.when(j + NBUF - 1 < _CPT)
            def _fire_next():
                pltpu.async_copy(emb_hbm.at[src_v.at[j + NBUF - 1]],
                                 rows_v.at[bp], gsem.at[bp])
        return carry

    lax.fori_loop(0, _CPT // NBUF, group, 0)
    pltpu.make_async_copy(rows_v.at[NBUF - 1],
                          acc_sh.at[dst_v.at[_CPT - 1]],
                          ssem.at[NBUF - 1]).wait()
    plsc.subcore_barrier()

    pltpu.sync_copy(acc_sh.at[pl.ds(sid * _RPT, _RPT)],
                    out_hbm.at[cid, pl.ds(sid * _RPT, _RPT)])


def _head_body(e_ref, p_ref, wc_ref, bc_ref, wo_ref, bo_ref, o_ref):
    h = e_ref[...] + p_ref[0] + p_ref[1]
    e2 = jnp.maximum(
        jnp.dot(h, wc_ref[...], preferred_element_type=jnp.float32)
        + bc_ref[...], 0.0)
    z = jnp.dot(e2, wo_ref[...], preferred_element_type=jnp.float32)
    o_ref[...] = jax.nn.sigmoid(z + bo_ref[...])


def _head(node_emb, partials, wc_t, bc2, wo_t, bo2):
    n, h = node_emb.shape
    bn = 1000
    return pl.pallas_call(
        _head_body,
        grid=(n // bn,),
        in_specs=[
            pl.BlockSpec((bn, h), lambda i: (i, 0)),
            pl.BlockSpec((NC, bn, h), lambda i: (0, i, 0)),
            pl.BlockSpec((h, h), lambda i: (0, 0)),
            pl.BlockSpec((1, h), lambda i: (0, 0)),
            pl.BlockSpec((h, 1), lambda i: (0, 0)),
            pl.BlockSpec((1, 1), lambda i: (0, 0)),
        ],
        out_specs=pl.BlockSpec((bn, 1), lambda i: (i, 0)),
        out_shape=jax.ShapeDtypeStruct((n, 1), jnp.float32),
    )(node_emb, partials, wc_t, bc2, wo_t, bo2)


@functools.partial(
    pl.kernel,
    out_type=jax.ShapeDtypeStruct((_PPAD,), jnp.float32),
    mesh=_MESH,
    scratch_types=[
        pltpu.VMEM((_N,), jnp.float32),
        pltpu.VMEM((_QPT,), jnp.int32),
        pltpu.VMEM((_QPT,), jnp.float32),
    ],
    compiler_params=pltpu.CompilerParams(needs_layout_passes=False),
)
def _sc_post_gather(s_hbm, pm_hbm, out_hbm, s_v, pm_v, o_v):
    cid = lax.axis_index("c")
    sid = lax.axis_index("s")
    wid = cid * NS + sid
    pltpu.sync_copy(s_hbm, s_v)
    pltpu.sync_copy(pm_hbm.at[pl.ds(wid * _QPT, _QPT)], pm_v)

    def gloop(i, carry):
        idx = pm_v[pl.ds(i * LANES, LANES)]
        o_v[pl.ds(i * LANES, LANES)] = plsc.load_gather(s_v, [idx])
        return carry

    lax.fori_loop(0, _QPT // LANES, gloop, 0)
    pltpu.sync_copy(o_v, out_hbm.at[pl.ds(wid * _QPT, _QPT)])


def kernel(node_features, edge_index, post_mask,
           W_enc, b_enc, W_conv, b_conv, W_out, b_out):
    n, d = node_features.shape
    h = W_enc.shape[0]
    e = edge_index.shape[1]
    p = post_mask.shape[0]

    node_emb = _encoder(node_features, W_enc.T, b_enc.reshape(1, h))

    src3d = edge_index[0].reshape(NW, _CPT, CHUNK)
    dst3d = edge_index[1].reshape(NW, _CPT, CHUNK)
    partials = _sc_messages(node_emb, src3d, dst3d)

    s = _head(node_emb, partials,
              W_conv.T, b_conv.reshape(1, h),
              W_out.T, b_out.reshape(1, 1))

    pm_pad = jnp.zeros((_PPAD,), jnp.int32).at[:p].set(post_mask)
    out = _sc_post_gather(s.reshape(-1), pm_pad)
    return out[:p]


# speedup vs baseline: 16.4702x; 1.0649x over previous
import functools

import jax
import jax.numpy as jnp
from jax import lax
from jax.experimental import pallas as pl
from jax.experimental.pallas import tpu as pltpu
from jax.experimental.pallas import tpu_sc as plsc

NC, NS, LANES = 2, 16, 16
NW = NC * NS
CHUNK = 80

_N, _D, _H, _E = 10000, 128, 32, 320000
_NPAD = 10240
_EPT = _E // NW
_CPT = _EPT // CHUNK
_RPT = _NPAD // NS
_ZR = 128
_PPAD = 5120
_PPT = _PPAD // NS
_EPP = _PPAD // NW
NBUF = 5


def _enc_body(x_ref, w_ref, b_ref, o_ref):
    acc = jnp.dot(x_ref[...], w_ref[...], preferred_element_type=jnp.float32)
    o_ref[...] = jnp.maximum(acc + b_ref[...], 0.0)


def _encoder(x, w_t, b2):
    n, d = x.shape
    h = w_t.shape[1]
    bn = 1000
    return pl.pallas_call(
        _enc_body,
        grid=(n // bn,),
        in_specs=[
            pl.BlockSpec((bn, d), lambda i: (i, 0)),
            pl.BlockSpec((d, h), lambda i: (0, 0)),
            pl.BlockSpec((1, h), lambda i: (0, 0)),
        ],
        out_specs=pl.BlockSpec((bn, h), lambda i: (i, 0)),
        out_shape=jax.ShapeDtypeStruct((n, h), jnp.float32),
    )(x, w_t, b2)


_MESH = plsc.VectorSubcoreMesh(core_axis_name="c", subcore_axis_name="s")


@functools.partial(
    pl.kernel,
    out_type=(
        jax.ShapeDtypeStruct((NC, _PPAD, _H), jnp.float32),
        jax.ShapeDtypeStruct((_PPAD, _H), jnp.float32),
    ),
    mesh=_MESH,
    scratch_types=[
        pltpu.VMEM((_CPT, CHUNK), jnp.int32),
        pltpu.VMEM((_CPT, CHUNK), jnp.int32),
        pltpu.VMEM((NBUF, CHUNK, _H), jnp.float32),
        pltpu.VMEM((_ZR, _H), jnp.float32),
        pltpu.VMEM_SHARED((_NPAD, _H), jnp.float32),
        pltpu.VMEM((_PPT,), jnp.int32),
        pltpu.VMEM((_PPT, _H), jnp.float32),
        pltpu.VMEM((_EPP,), jnp.int32),
        pltpu.VMEM((_EPP, _H), jnp.float32),
        pltpu.SemaphoreType.DMA((NBUF,)),
        pltpu.SemaphoreType.DMA((NBUF,)),
        pltpu.SemaphoreType.DMA,
    ],
    compiler_params=pltpu.CompilerParams(use_tc_tiling_on_sc=False),
)
def _sc_messages(emb_hbm, src_hbm, dst_hbm, pm_hbm, gp_hbm, ge_hbm,
                 src_v, dst_v, rows_v, z_v, acc_sh,
                 pmp_v, pout_v, pme_v, eout_v, gsem, ssem, psem):
    cid = lax.axis_index("c")
    sid = lax.axis_index("s")
    wid = cid * NS + sid

    zero16 = jnp.zeros((16,), jnp.float32)

    def zfill(i, carry):
        z_v[i, pl.ds(0, 16)] = zero16
        z_v[i, pl.ds(16, 16)] = zero16
        return carry

    lax.fori_loop(0, _ZR, zfill, 0)

    def zcopy(r, carry):
        pltpu.sync_copy(z_v, acc_sh.at[pl.ds(sid * _RPT + r * _ZR, _ZR)])
        return carry

    lax.fori_loop(0, _RPT // _ZR, zcopy, 0)

    pltpu.sync_copy(src_hbm.at[wid], src_v)
    pltpu.sync_copy(dst_hbm.at[wid], dst_v)
    pltpu.sync_copy(pm_hbm.at[pl.ds(sid * _PPT, _PPT)], pmp_v)
    pltpu.sync_copy(pm_hbm.at[pl.ds(wid * _EPP, _EPP)], pme_v)
    plsc.subcore_barrier()

    for b in range(NBUF - 1):
        pltpu.async_copy(emb_hbm.at[src_v.at[b]], rows_v.at[b], gsem.at[b])

    def group(g, carry):
        for b in range(NBUF):
            j = g * NBUF + b
            bp = (b - 1) % NBUF
            pltpu.make_async_copy(
                emb_hbm.at[src_v.at[j]], rows_v.at[b], gsem.at[b]).wait()
            pltpu.async_copy(
                rows_v.at[b], acc_sh.at[dst_v.at[j]], ssem.at[b], add=True)

            @pl.when(j >= 1)
            def _wait_prev():
                pltpu.make_async_copy(
                    rows_v.at[bp], acc_sh.at[dst_v.at[j - 1]],
                    ssem.at[bp]).wait()

            @pl.when(j + NBUF - 1 < _CPT)
            def _fire_next():
                pltpu.async_copy(emb_hbm.at[src_v.at[j + NBUF - 1]],
                                 rows_v.at[bp], gsem.at[bp])
        return carry

    lax.fori_loop(0, _CPT // NBUF, group, 0)
    pltpu.make_async_copy(rows_v.at[NBUF - 1],
                          acc_sh.at[dst_v.at[_CPT - 1]],
                          ssem.at[NBUF - 1]).wait()
    plsc.subcore_barrier()

    for k in range(_PPT // CHUNK):
        pltpu.async_copy(acc_sh.at[pmp_v.at[pl.ds(k * CHUNK, CHUNK)]],
                         pout_v.at[pl.ds(k * CHUNK, CHUNK)], psem).wait()
    for k in range(_EPP // CHUNK):
        pltpu.async_copy(emb_hbm.at[pme_v.at[pl.ds(k * CHUNK, CHUNK)]],
                         eout_v.at[pl.ds(k * CHUNK, CHUNK)], psem).wait()
    pltpu.sync_copy(pout_v, gp_hbm.at[cid, pl.ds(sid * _PPT, _PPT)])
    pltpu.sync_copy(eout_v, ge_hbm.at[pl.ds(wid * _EPP, _EPP)])


def _head_body(e_ref, p_ref, wc_ref, bc_ref, wo_ref, bo_ref, o_ref):
    h = e_ref[...] + p_ref[0] + p_ref[1]
    e2 = jnp.maximum(
        jnp.dot(h, wc_ref[...], preferred_element_type=jnp.float32)
        + bc_ref[...], 0.0)
    z = jnp.dot(e2, wo_ref[...], preferred_element_type=jnp.float32)
    o_ref[...] = jax.nn.sigmoid(z + bo_ref[...])


def _head(ge, gp, wc_t, bc2, wo_t, bo2):
    n, h = ge.shape
    bn = 512
    return pl.pallas_call(
        _head_body,
        grid=(n // bn,),
        in_specs=[
            pl.BlockSpec((bn, h), lambda i: (i, 0)),
            pl.BlockSpec((NC, bn, h), lambda i: (0, i, 0)),
            pl.BlockSpec((h, h), lambda i: (0, 0)),
            pl.BlockSpec((1, h), lambda i: (0, 0)),
            pl.BlockSpec((h, 1), lambda i: (0, 0)),
            pl.BlockSpec((1, 1), lambda i: (0, 0)),
        ],
        out_specs=pl.BlockSpec((bn, 1), lambda i: (i, 0)),
        out_shape=jax.ShapeDtypeStruct((n, 1), jnp.float32),
    )(ge, gp, wc_t, bc2, wo_t, bo2)


def kernel(node_features, edge_index, post_mask,
           W_enc, b_enc, W_conv, b_conv, W_out, b_out):
    n, d = node_features.shape
    h = W_enc.shape[0]
    p = post_mask.shape[0]

    node_emb = _encoder(node_features, W_enc.T, b_enc.reshape(1, h))

    src3d = edge_index[0].reshape(NW, _CPT, CHUNK)
    dst3d = edge_index[1].reshape(NW, _CPT, CHUNK)
    pm_pad = jnp.zeros((_PPAD,), jnp.int32).at[:p].set(post_mask)
    gp, ge = _sc_messages(node_emb, src3d, dst3d, pm_pad)

    s = _head(ge, gp,
              W_conv.T, b_conv.reshape(1, h),
              W_out.T, b_out.reshape(1, 1))
    return s.reshape(-1)[:p]


# speedup vs baseline: 21.0512x; 1.2781x over previous
import functools

import jax
import jax.numpy as jnp
from jax import lax
from jax.experimental import pallas as pl
from jax.experimental.pallas import tpu as pltpu
from jax.experimental.pallas import tpu_sc as plsc

NC, NS, LANES = 2, 16, 16
NW = NC * NS
CHUNK = 125

_N, _D, _H, _E = 10000, 128, 32, 320000
_NPAD = 10240
_EPT = _E // NW
_CPT = _EPT // CHUNK
_RPT = _NPAD // NS
_ZR = 128
_PPAD = 5120
_PPT = _PPAD // NS
_EPP = _PPAD // NW
NBUF = 8


def _enc_body(x_ref, w_ref, b_ref, o_ref):
    acc = jnp.dot(x_ref[...], w_ref[...], preferred_element_type=jnp.float32)
    o_ref[...] = jnp.maximum(acc + b_ref[...], 0.0)


def _encoder(x, w_t, b2):
    n, d = x.shape
    h = w_t.shape[1]
    bn = n // 2
    return pl.pallas_call(
        _enc_body,
        grid=(n // bn,),
        in_specs=[
            pl.BlockSpec((bn, d), lambda i: (i, 0)),
            pl.BlockSpec((d, h), lambda i: (0, 0)),
            pl.BlockSpec((1, h), lambda i: (0, 0)),
        ],
        out_specs=pl.BlockSpec((bn, h), lambda i: (i, 0)),
        out_shape=jax.ShapeDtypeStruct((n, h), jnp.float32),
    )(x, w_t, b2)


_MESH = plsc.VectorSubcoreMesh(core_axis_name="c", subcore_axis_name="s")


@functools.partial(
    pl.kernel,
    out_type=(
        jax.ShapeDtypeStruct((NC, _PPAD, _H), jnp.float32),
        jax.ShapeDtypeStruct((_PPAD, _H), jnp.float32),
    ),
    mesh=_MESH,
    scratch_types=[
        pltpu.VMEM((_CPT, CHUNK), jnp.int32),
        pltpu.VMEM((_CPT, CHUNK), jnp.int32),
        pltpu.VMEM((NBUF, CHUNK, _H), jnp.float32),
        pltpu.VMEM((_ZR, _H), jnp.float32),
        pltpu.VMEM_SHARED((_NPAD, _H), jnp.float32),
        pltpu.VMEM((_PPT,), jnp.int32),
        pltpu.VMEM((_PPT, _H), jnp.float32),
        pltpu.VMEM((_EPP,), jnp.int32),
        pltpu.VMEM((_EPP, _H), jnp.float32),
        pltpu.SemaphoreType.DMA((NBUF,)),
        pltpu.SemaphoreType.DMA((NBUF,)),
        pltpu.SemaphoreType.DMA,
    ],
    compiler_params=pltpu.CompilerParams(use_tc_tiling_on_sc=False),
)
def _sc_messages(emb_hbm, ei_hbm, pm_hbm, gp_hbm, ge_hbm,
                 src_v, dst_v, rows_v, z_v, acc_sh,
                 pmp_v, pout_v, pme_v, eout_v, gsem, ssem, psem):
    cid = lax.axis_index("c")
    sid = lax.axis_index("s")
    wid = cid * NS + sid

    cp_src = pltpu.async_copy(ei_hbm.at[0, wid], src_v, psem)
    cp_dst = pltpu.async_copy(ei_hbm.at[1, wid], dst_v, psem)
    cp_pmp = pltpu.async_copy(pm_hbm.at[pl.ds(sid * _PPT, _PPT)], pmp_v, psem)
    cp_pme = pltpu.async_copy(pm_hbm.at[pl.ds(wid * _EPP, _EPP)], pme_v, psem)

    zero16 = jnp.zeros((16,), jnp.float32)

    def zfill(i, carry):
        z_v[i, pl.ds(0, 16)] = zero16
        z_v[i, pl.ds(16, 16)] = zero16
        return carry

    lax.fori_loop(0, _ZR, zfill, 0)

    def zcopy(r, carry):
        pltpu.sync_copy(z_v, acc_sh.at[pl.ds(sid * _RPT + r * _ZR, _ZR)])
        return carry

    lax.fori_loop(0, _RPT // _ZR, zcopy, 0)
    cp_src.wait()
    cp_dst.wait()
    cp_pmp.wait()
    cp_pme.wait()
    plsc.subcore_barrier()

    for b in range(NBUF - 1):
        pltpu.async_copy(emb_hbm.at[src_v.at[b]], rows_v.at[b], gsem.at[b])

    def group(g, carry):
        for b in range(NBUF):
            j = g * NBUF + b
            bp = (b - 1) % NBUF
            pltpu.make_async_copy(
                emb_hbm.at[src_v.at[j]], rows_v.at[b], gsem.at[b]).wait()
            pltpu.async_copy(
                rows_v.at[b], acc_sh.at[dst_v.at[j]], ssem.at[b], add=True)

            @pl.when(j >= 1)
            def _wait_prev():
                pltpu.make_async_copy(
                    rows_v.at[bp], acc_sh.at[dst_v.at[j - 1]],
                    ssem.at[bp]).wait()

            @pl.when(j + NBUF - 1 < _CPT)
            def _fire_next():
                pltpu.async_copy(emb_hbm.at[src_v.at[j + NBUF - 1]],
                                 rows_v.at[bp], gsem.at[bp])
        return carry

    lax.fori_loop(0, _CPT // NBUF, group, 0)
    pltpu.make_async_copy(rows_v.at[NBUF - 1],
                          acc_sh.at[dst_v.at[_CPT - 1]],
                          ssem.at[NBUF - 1]).wait()
    plsc.subcore_barrier()

    for k in range(4):
        pltpu.async_copy(acc_sh.at[pmp_v.at[pl.ds(k * 80, 80)]],
                         pout_v.at[pl.ds(k * 80, 80)], gsem.at[k])
    for k in range(2):
        pltpu.async_copy(emb_hbm.at[pme_v.at[pl.ds(k * 80, 80)]],
                         eout_v.at[pl.ds(k * 80, 80)], gsem.at[4 + k])
    for k in range(4):
        pltpu.make_async_copy(acc_sh.at[pmp_v.at[pl.ds(k * 80, 80)]],
                              pout_v.at[pl.ds(k * 80, 80)], gsem.at[k]).wait()
    for k in range(2):
        pltpu.make_async_copy(emb_hbm.at[pme_v.at[pl.ds(k * 80, 80)]],
                              eout_v.at[pl.ds(k * 80, 80)],
                              gsem.at[4 + k]).wait()
    pltpu.sync_copy(pout_v, gp_hbm.at[cid, pl.ds(sid * _PPT, _PPT)])
    pltpu.sync_copy(eout_v, ge_hbm.at[pl.ds(wid * _EPP, _EPP)])


def _head_body(e_ref, p_ref, wc_ref, bc_ref, wo_ref, bo_ref, o_ref):
    h = e_ref[...] + p_ref[0] + p_ref[1]
    e2 = jnp.maximum(
        jnp.dot(h, wc_ref[...], preferred_element_type=jnp.float32)
        + bc_ref[...], 0.0)
    z = jnp.dot(e2, wo_ref[...], preferred_element_type=jnp.float32)
    o_ref[...] = jax.nn.sigmoid(z + bo_ref[...])


def _head(ge, gp, wc_t, bc2, wo_t, bo2):
    n, h = ge.shape
    return pl.pallas_call(
        _head_body,
        grid=(1,),
        in_specs=[
            pl.BlockSpec((n, h), lambda i: (0, 0)),
            pl.BlockSpec((NC, n, h), lambda i: (0, 0, 0)),
            pl.BlockSpec((h, h), lambda i: (0, 0)),
            pl.BlockSpec((1, h), lambda i: (0, 0)),
            pl.BlockSpec((h, 1), lambda i: (0, 0)),
            pl.BlockSpec((1, 1), lambda i: (0, 0)),
        ],
        out_specs=pl.BlockSpec((n, 1), lambda i: (0, 0)),
        out_shape=jax.ShapeDtypeStruct((n, 1), jnp.float32),
    )(ge, gp, wc_t, bc2, wo_t, bo2)


def kernel(node_features, edge_index, post_mask,
           W_enc, b_enc, W_conv, b_conv, W_out, b_out):
    n, d = node_features.shape
    h = W_enc.shape[0]
    p = post_mask.shape[0]

    node_emb = _encoder(node_features, W_enc.T, b_enc.reshape(1, h))

    ei4 = edge_index.reshape(2, NW, _CPT, CHUNK)
    pm_pad = jnp.pad(post_mask, (0, _PPAD - p))
    gp, ge = _sc_messages(node_emb, ei4, pm_pad)

    s = _head(ge, gp,
              W_conv.T, b_conv.reshape(1, h),
              W_out.T, b_out.reshape(1, 1))
    return s.reshape(-1)[:p]
